# baseline (speedup 1.0000x reference)
import jax
import jax.numpy as jnp
from jax.experimental import pallas as pl

_NV = 1024
_LD = 64
_B = 16
_HW = 32 * 32


def _vq_body(z_ref, w_ref, zq_ref, idx_ref, loss_ref):
    zb4 = z_ref[0]
    zp = jnp.transpose(zb4, (1, 2, 0)).reshape(_HW, _LD)
    w = w_ref[...]

    zsq = jnp.sum(zp * zp, axis=1, keepdims=True)
    wt = w.T
    wsq = jnp.sum(wt * wt, axis=0, keepdims=True)
    mm2 = jax.lax.dot_general(zp, w + w, (((1,), (1,)), ((), ())),
                              preferred_element_type=jnp.float32)
    d = (zsq + wsq) - mm2

    dmin = jnp.min(d, axis=1, keepdims=True)
    ids = jax.lax.broadcasted_iota(jnp.int32, d.shape, 1)
    idxk = jnp.min(jnp.where(d == dmin, ids, jnp.int32(_NV)),
                   axis=1, keepdims=True)

    oh = (ids == idxk).astype(jnp.bfloat16)
    zq = jax.lax.dot_general(oh, w, (((1,), (0,)), ((), ())),
                             preferred_element_type=jnp.float32)

    loss_ref[0] = (zq - zp) ** 2
    zq_ref[0] = jnp.transpose(zq.reshape(32, 32, _LD), (2, 0, 1))
    idx_ref[0] = idxk.T


def kernel(z, W):
    zq4, idx3, loss3 = pl.pallas_call(
        _vq_body,
        grid=(_B,),
        in_specs=[
            pl.BlockSpec((1, _LD, 32, 32), lambda b: (b, 0, 0, 0)),
            pl.BlockSpec((_NV, _LD), lambda b: (0, 0)),
        ],
        out_specs=[
            pl.BlockSpec((1, _LD, 32, 32), lambda b: (b, 0, 0, 0)),
            pl.BlockSpec((1, 1, _HW), lambda b: (b, 0, 0)),
            pl.BlockSpec((1, _HW, _LD), lambda b: (b, 0, 0)),
        ],
        out_shape=[
            jax.ShapeDtypeStruct((_B, _LD, 32, 32), jnp.float32),
            jax.ShapeDtypeStruct((_B, 1, _HW), jnp.int32),
            jax.ShapeDtypeStruct((_B, _HW, _LD), jnp.float32),
        ],
    )(z, W)
    min_encoding_indices = idx3.reshape(_B * _HW)
    loss = loss3.reshape(_B, 32, 32, _LD)
    return (zq4, min_encoding_indices, loss)


# speedup vs baseline: 1.3861x; 1.3861x over previous
import jax
import jax.numpy as jnp
from jax.experimental import pallas as pl

_NV = 1024
_LD = 64
_B = 16
_HW = 32 * 32


def _vq_body(z_ref, w_ref, zq_ref, idx_ref, loss_ref):
    zb = z_ref[0]
    zp = zb.T
    w = w_ref[...]

    zsq = jnp.sum(zp * zp, axis=1, keepdims=True)
    wt = w.T
    wsq = jnp.sum(wt * wt, axis=0, keepdims=True)
    mm2 = jax.lax.dot_general(zp, w + w, (((1,), (1,)), ((), ())),
                              preferred_element_type=jnp.float32)
    d = (zsq + wsq) - mm2

    dmin = jnp.min(d, axis=1, keepdims=True)
    ids = jax.lax.broadcasted_iota(jnp.int32, d.shape, 1)
    idxk = jnp.min(jnp.where(d == dmin, ids, jnp.int32(_NV)),
                   axis=1, keepdims=True)

    oh = (ids == idxk).astype(jnp.bfloat16)
    zq = jax.lax.dot_general(oh, w, (((1,), (0,)), ((), ())),
                             preferred_element_type=jnp.float32)

    loss_ref[0] = (zq - zp) ** 2
    zq_ref[0] = zq.T
    idx_ref[0] = idxk.T


def kernel(z, W):
    z3 = z.reshape(_B, _LD, _HW)
    zq3, idx3, loss3 = pl.pallas_call(
        _vq_body,
        grid=(_B,),
        in_specs=[
            pl.BlockSpec((1, _LD, _HW), lambda b: (b, 0, 0)),
            pl.BlockSpec((_NV, _LD), lambda b: (0, 0)),
        ],
        out_specs=[
            pl.BlockSpec((1, _LD, _HW), lambda b: (b, 0, 0)),
            pl.BlockSpec((1, 1, _HW), lambda b: (b, 0, 0)),
            pl.BlockSpec((1, _HW, _LD), lambda b: (b, 0, 0)),
        ],
        out_shape=[
            jax.ShapeDtypeStruct((_B, _LD, _HW), jnp.float32),
            jax.ShapeDtypeStruct((_B, 1, _HW), jnp.int32),
            jax.ShapeDtypeStruct((_B, _HW, _LD), jnp.float32),
        ],
    )(z3, W)
    z_q_out = zq3.reshape(_B, _LD, 32, 32)
    min_encoding_indices = idx3.reshape(_B * _HW)
    loss = loss3.reshape(_B, 32, 32, _LD)
    return (z_q_out, min_encoding_indices, loss)


# speedup vs baseline: 1.4800x; 1.0678x over previous
import jax
import jax.numpy as jnp
from jax.experimental import pallas as pl

_NV = 1024
_LD = 64
_B = 16
_HW = 32 * 32
_BB = 2
_M = _BB * _HW


def _argmin_rows(d):
    m = d.shape[0]
    val = d
    idx = jax.lax.broadcasted_iota(jnp.int32, d.shape, 1)
    width = d.shape[1]
    while width > 128:
        half = width // 2
        vl, vr = val[:, :half], val[:, half:]
        il, ir = idx[:, :half], idx[:, half:]
        take = vr < vl
        val = jnp.where(take, vr, vl)
        idx = jnp.where(take, ir, il)
        width = half
    dmin = jnp.min(val, axis=1, keepdims=True)
    return jnp.min(jnp.where(val == dmin, idx, jnp.int32(_NV)),
                   axis=1, keepdims=True)


def _vq_body(z_ref, w_ref, zq_ref, idx_ref, loss_ref):
    zb = z_ref[...]
    zp = jnp.transpose(zb, (0, 2, 1)).reshape(_M, _LD)
    w = w_ref[...]

    zsq = jnp.sum(zp * zp, axis=1, keepdims=True)
    wt = w.T
    wsq = jnp.sum(wt * wt, axis=0, keepdims=True)
    mm2 = jax.lax.dot_general(zp, w + w, (((1,), (1,)), ((), ())),
                              preferred_element_type=jnp.float32)
    d = (zsq + wsq) - mm2

    idxk = _argmin_rows(d)

    ids = jax.lax.broadcasted_iota(jnp.int32, d.shape, 1)
    oh = (ids == idxk).astype(jnp.bfloat16)
    zq = jax.lax.dot_general(oh, w.astype(jnp.bfloat16),
                             (((1,), (0,)), ((), ())),
                             preferred_element_type=jnp.float32)

    loss_ref[...] = ((zq - zp) ** 2).reshape(_BB, _HW, _LD)
    zq_ref[0] = zq[:_HW].T
    zq_ref[1] = zq[_HW:].T
    idx_ref[0] = idxk.T


def kernel(z, W):
    z3 = z.reshape(_B, _LD, _HW)
    nsteps = _B // _BB
    zq3, idx3, loss3 = pl.pallas_call(
        _vq_body,
        grid=(nsteps,),
        in_specs=[
            pl.BlockSpec((_BB, _LD, _HW), lambda b: (b, 0, 0)),
            pl.BlockSpec((_NV, _LD), lambda b: (0, 0)),
        ],
        out_specs=[
            pl.BlockSpec((_BB, _LD, _HW), lambda b: (b, 0, 0)),
            pl.BlockSpec((1, 1, _M), lambda b: (b, 0, 0)),
            pl.BlockSpec((_BB, _HW, _LD), lambda b: (b, 0, 0)),
        ],
        out_shape=[
            jax.ShapeDtypeStruct((_B, _LD, _HW), jnp.float32),
            jax.ShapeDtypeStruct((nsteps, 1, _M), jnp.int32),
            jax.ShapeDtypeStruct((_B, _HW, _LD), jnp.float32),
        ],
    )(z3, W)
    z_q_out = zq3.reshape(_B, _LD, 32, 32)
    min_encoding_indices = idx3.reshape(_B * _HW)
    loss = loss3.reshape(_B, 32, 32, _LD)
    return (z_q_out, min_encoding_indices, loss)
